# baseline (speedup 1.0000x reference)
import functools

import jax
import jax.numpy as jnp
from jax import lax
from jax.experimental import pallas as pl
from jax.experimental.pallas import tpu as pltpu
from jax.experimental.pallas import tpu_sc as plsc

N = 10000
E = 320000
H = 128
L = 4
K = 4
G = 512
NZ = 1000

NC = 2
NS = 16
NW = NC * NS
CP = 16
EPWP = 10016
EP = NW * EPWP
NCHW = EPWP // CP
FPC = 11 * CP
RPT = 624

R = 2000
GRID_N = N // R

_f32 = jnp.float32
_HOPS = H // 16


def _make_edge_kernel(k):
  mesh = plsc.VectorSubcoreMesh(core_axis_name="c", subcore_axis_name="s")
  out_type = [
      jax.ShapeDtypeStruct((NC, N, H), _f32),
      jax.ShapeDtypeStruct((NC, N * 4), _f32),
  ]
  scratch_types = (
      [pltpu.VMEM((FPC,), _f32) for _ in range(2)]
      + [pltpu.VMEM((CP,), jnp.int32) for _ in range(6)]
      + [pltpu.VMEM((CP,), _f32) for _ in range(2)]
      + [pltpu.VMEM((CP, H), _f32) for _ in range(2 * k)]
      + [pltpu.VMEM((CP, H), _f32) for _ in range(2)]
      + [pltpu.VMEM_SHARED((N, H), _f32),
         pltpu.VMEM_SHARED((N * 4,), _f32)]
      + [pltpu.SemaphoreType.DMA for _ in range(8)]
  )

  @functools.partial(pl.kernel, mesh=mesh, out_type=out_type,
                     scratch_types=scratch_types)
  def edge_kernel(*refs):
    epack = refs[0]
    h_hs = refs[1:1 + k]
    zero_nh = refs[1 + k]
    zero_n4 = refs[2 + k]
    agg_out = refs[3 + k]
    w_out = refs[4 + k]
    s = 5 + k
    ebufs = refs[s:s + 2]
    src_i = refs[s + 2:s + 4]
    dst_i = refs[s + 4:s + 6]
    cidx_i = refs[s + 6:s + 8]
    pbuf = refs[s + 8:s + 10]
    gb = refs[s + 10:s + 10 + 2 * k]
    g = [gb[:k], gb[k:]]
    msg = refs[s + 10 + 2 * k:s + 12 + 2 * k]
    agg_sh = refs[s + 12 + 2 * k]
    w_sh = refs[s + 13 + 2 * k]
    sems = refs[s + 14 + 2 * k:]
    lsems, gsems, msems, psems = sems[0:2], sems[2:4], sems[4:6], sems[6:8]

    cid = lax.axis_index("c")
    sid = lax.axis_index("s")
    base = (cid * NS + sid) * NCHW

    @pl.when(sid == 0)
    def _init():
      pltpu.sync_copy(zero_nh, agg_sh)
      pltpu.sync_copy(zero_n4, w_sh)

    plsc.subcore_barrier()

    def lin_start(c, p):
      pltpu.async_copy(epack.at[pl.ds((base + c) * FPC, FPC)], ebufs[p],
                       lsems[p])

    def lin_wait(c, p):
      pltpu.make_async_copy(epack.at[pl.ds((base + c) * FPC, FPC)], ebufs[p],
                            lsems[p]).wait()

    def decode_src(p):
      for t in range(CP // 16):
        src_i[p][pl.ds(16 * t, 16)] = ebufs[p][pl.ds(16 * t, 16)].astype(
            jnp.int32)

    def gat_start(p):
      for j in range(k):
        pltpu.async_copy(h_hs[j].at[src_i[p]], g[p][j], gsems[p])

    def gat_wait(p):
      for j in range(k):
        pltpu.make_async_copy(h_hs[j].at[src_i[p]], g[p][j], gsems[p]).wait()

    def sct_wait(p):
      del p

    def compute_and_scatter(p):
      eb = ebufs[p]
      for t in range(CP // 16):
        sl = pl.ds(16 * t, 16)
        d = eb[pl.ds(CP + 16 * t, 16)].astype(jnp.int32)
        b = eb[pl.ds(2 * CP + 16 * t, 16)].astype(jnp.int32)
        dst_i[p][sl] = d
        cidx_i[p][sl] = d * 4 + b
        pv = eb[pl.ds(7 * CP + 16 * t, 16)]
        for j in range(1, k):
          pv = pv + eb[pl.ds((7 + j) * CP + 16 * t, 16)]
        pbuf[p][sl] = pv

      for gi in range(CP // 16):
        e0 = gi * 16
        ea_vecs = [eb[pl.ds((3 + j) * CP + e0, 16)] for j in range(k)]
        for lane in range(16):
          e = e0 + lane
          s0 = ea_vecs[0][lane]
          acc = [g[p][0][e, pl.ds(16 * i, 16)] * s0 for i in range(_HOPS)]
          for j in range(1, k):
            sj = ea_vecs[j][lane]
            for i in range(_HOPS):
              acc[i] = acc[i] + g[p][j][e, pl.ds(16 * i, 16)] * sj
          for i in range(_HOPS):
            msg[p][e, pl.ds(16 * i, 16)] = acc[i]

      pltpu.sync_copy(msg[p], agg_sh.at[dst_i[p]], add=True)
      pltpu.sync_copy(pbuf[p], w_sh.at[cidx_i[p]], add=True)

    def pair(c2, carry):
      c0 = 2 * c2
      lin_start(c0, 0)
      lin_start(c0 + 1, 1)
      lin_wait(c0, 0)
      decode_src(0)
      gat_start(0)
      lin_wait(c0 + 1, 1)
      decode_src(1)
      gat_start(1)
      gat_wait(0)
      compute_and_scatter(0)
      gat_wait(1)
      compute_and_scatter(1)
      return carry

    lax.fori_loop(0, NCHW // 2, pair, 0)

    plsc.subcore_barrier()
    r0 = sid * RPT
    pltpu.sync_copy(agg_sh.at[pl.ds(r0, RPT)],
                    agg_out.at[cid, pl.ds(r0, RPT)])

    @pl.when(sid == 0)
    def _wout():
      pltpu.sync_copy(agg_sh.at[pl.ds(NS * RPT, N - NS * RPT)],
                      agg_out.at[cid, pl.ds(NS * RPT, N - NS * RPT)])
      pltpu.sync_copy(w_sh, w_out.at[cid])

  return edge_kernel


_EDGE_KERNELS = {k: _make_edge_kernel(k) for k in range(1, K + 1)}


def _embed_body(z_ref, x_ref, rd_ref, zt_ref, rdw_ref, rdb_ref, iwt_ref,
                iwb_ref, ib_ref, o_ref):
  oh = (z_ref[...] == lax.broadcasted_iota(jnp.int32, (1, NZ), 1)).astype(_f32)
  ze = jnp.dot(oh, zt_ref[...], preferred_element_type=_f32)
  a8 = ze + rd_ref[...] * rdw_ref[...] + rdb_ref[...]
  o_ref[...] = (jnp.dot(a8, iwt_ref[...], preferred_element_type=_f32)
                + jnp.dot(x_ref[...], iwb_ref[...], preferred_element_type=_f32)
                + ib_ref[...])


def _embed(z2, x, rd, z_table, rd_W, rd_b2, init_Wt, init_Wb, init_b2):
  return pl.pallas_call(
      _embed_body,
      grid=(GRID_N,),
      in_specs=[
          pl.BlockSpec((R, 1), lambda i: (i, 0)),
          pl.BlockSpec((R, H - 8), lambda i: (i, 0)),
          pl.BlockSpec((R, 1), lambda i: (i, 0)),
          pl.BlockSpec((NZ, 8), lambda i: (0, 0)),
          pl.BlockSpec((1, 8), lambda i: (0, 0)),
          pl.BlockSpec((1, 8), lambda i: (0, 0)),
          pl.BlockSpec((8, H), lambda i: (0, 0)),
          pl.BlockSpec((H - 8, H), lambda i: (0, 0)),
          pl.BlockSpec((1, H), lambda i: (0, 0)),
      ],
      out_specs=pl.BlockSpec((R, H), lambda i: (i, 0)),
      out_shape=jax.ShapeDtypeStruct((N, H), _f32),
  )(z2, x, rd, z_table, rd_W, rd_b2, init_Wt, init_Wb, init_b2)


def _mlp_common(h_ref, agg_ref, w_ref, btab_ref, scale_ref, w1_ref, b1_ref,
                w2_ref, b2_ref):
  a = h_ref[...] * scale_ref[...] + agg_ref[0] + agg_ref[1]
  wsum = w_ref[0] + w_ref[1]
  a = a + jnp.dot(wsum, btab_ref[...], preferred_element_type=_f32)
  t = jnp.maximum(jnp.dot(a, w1_ref[...], preferred_element_type=_f32)
                  + b1_ref[...], 0.0)
  zz = jnp.dot(t, w2_ref[...], preferred_element_type=_f32) + b2_ref[...]
  mu = jnp.mean(zz, axis=-1, keepdims=True)
  var = jnp.mean((zz - mu) ** 2, axis=-1, keepdims=True)
  return (zz - mu) * lax.rsqrt(var + 1e-5)


def _mlp_mid_body(h_ref, agg_ref, w_ref, btab_ref, scale_ref, w1_ref, b1_ref,
                  w2_ref, b2_ref, vn_ref, batch_ref, o_ref):
  ln = _mlp_common(h_ref, agg_ref, w_ref, btab_ref, scale_ref, w1_ref, b1_ref,
                   w2_ref, b2_ref)
  oh = (batch_ref[...] == lax.broadcasted_iota(jnp.int32, (1, G), 1)
        ).astype(_f32)
  o_ref[...] = ln + jnp.dot(oh, vn_ref[...], preferred_element_type=_f32)


def _mlp_last_body(h_ref, agg_ref, w_ref, btab_ref, scale_ref, w1_ref, b1_ref,
                   w2_ref, b2_ref, ow_ref, ob_ref, o_ref):
  ln = _mlp_common(h_ref, agg_ref, w_ref, btab_ref, scale_ref, w1_ref, b1_ref,
                   w2_ref, b2_ref)
  o_ref[...] = jnp.maximum(
      jnp.dot(ln, ow_ref[...], preferred_element_type=_f32) + ob_ref[...], 0.0)


def _mlp_specs(extra):
  return [
      pl.BlockSpec((R, H), lambda i: (i, 0)),
      pl.BlockSpec((NC, R, H), lambda i: (0, i, 0)),
      pl.BlockSpec((NC, R, 4), lambda i: (0, i, 0)),
      pl.BlockSpec((4, H), lambda i: (0, 0)),
      pl.BlockSpec((1, H), lambda i: (0, 0)),
      pl.BlockSpec((H, 2 * H), lambda i: (0, 0)),
      pl.BlockSpec((1, 2 * H), lambda i: (0, 0)),
      pl.BlockSpec((2 * H, H), lambda i: (0, 0)),
      pl.BlockSpec((1, H), lambda i: (0, 0)),
  ] + extra


def _mlp_mid(h, agg2, w2, btab, scale_row, w1, b1, w2w, b2, vn, batch2):
  return pl.pallas_call(
      _mlp_mid_body,
      grid=(GRID_N,),
      in_specs=_mlp_specs([
          pl.BlockSpec((G, H), lambda i: (0, 0)),
          pl.BlockSpec((R, 1), lambda i: (i, 0)),
      ]),
      out_specs=pl.BlockSpec((R, H), lambda i: (i, 0)),
      out_shape=jax.ShapeDtypeStruct((N, H), _f32),
  )(h, agg2, w2, btab, scale_row, w1, b1, w2w, b2, vn, batch2)


def _mlp_last(h, agg2, w2, btab, scale_row, w1, b1, w2w, b2, out_W, out_b2):
  return pl.pallas_call(
      _mlp_last_body,
      grid=(GRID_N,),
      in_specs=_mlp_specs([
          pl.BlockSpec((H, H), lambda i: (0, 0)),
          pl.BlockSpec((1, H), lambda i: (0, 0)),
      ]),
      out_specs=pl.BlockSpec((R, H), lambda i: (i, 0)),
      out_shape=jax.ShapeDtypeStruct((N, H), _f32),
  )(h, agg2, w2, btab, scale_row, w1, b1, w2w, b2, out_W, out_b2)


def _vn_body(h_ref, vno_ref, batch_ref, w1_ref, b1_ref, w2_ref, b2_ref, o_ref):
  step = pl.program_id(0)
  oh = (batch_ref[...] == lax.broadcasted_iota(jnp.int32, (1, G), 1)
        ).astype(_f32)
  seg = lax.dot_general(oh, h_ref[...], (((0,), (0,)), ((), ())),
                        preferred_element_type=_f32)

  @pl.when(step == 0)
  def _():
    o_ref[...] = vno_ref[...] + seg

  @pl.when(step > 0)
  def _():
    o_ref[...] = o_ref[...] + seg

  @pl.when(step == GRID_N - 1)
  def _():
    acc = o_ref[...]
    t = jnp.maximum(jnp.dot(acc, w1_ref[...], preferred_element_type=_f32)
                    + b1_ref[...], 0.0)
    o_ref[...] = jnp.maximum(
        jnp.dot(t, w2_ref[...], preferred_element_type=_f32) + b2_ref[...], 0.0)


def _vn_update(h, vn_old, batch2, w1, b1, w2, b2):
  return pl.pallas_call(
      _vn_body,
      grid=(GRID_N,),
      in_specs=[
          pl.BlockSpec((R, H), lambda i: (i, 0)),
          pl.BlockSpec((G, H), lambda i: (0, 0)),
          pl.BlockSpec((R, 1), lambda i: (i, 0)),
          pl.BlockSpec((H, 2 * H), lambda i: (0, 0)),
          pl.BlockSpec((1, 2 * H), lambda i: (0, 0)),
          pl.BlockSpec((2 * H, H), lambda i: (0, 0)),
          pl.BlockSpec((1, H), lambda i: (0, 0)),
      ],
      out_specs=pl.BlockSpec((G, H), lambda i: (0, 0)),
      out_shape=jax.ShapeDtypeStruct((G, H), _f32),
  )(h, vn_old, batch2, w1, b1, w2, b2)


def kernel(x, z, edge_index, bond_feature, edge_attr, peripheral_attr, rd,
           pos, batch, z_table, rd_W, rd_b, init_W, init_b, gnn_W1, gnn_b1,
           gnn_W2, gnn_b2, gnn_eps, bond_table, vn_W1, vn_b1, vn_W2, vn_b2,
           out_W, out_b):
  del pos
  pad = EP - E
  pad_ids = jnp.arange(pad, dtype=jnp.int32) % N
  src = jnp.concatenate([edge_index[0].astype(jnp.int32), pad_ids])
  dst = jnp.concatenate([edge_index[1].astype(jnp.int32), pad_ids])
  bond = jnp.concatenate([bond_feature.astype(jnp.int32),
                          jnp.zeros((pad,), jnp.int32)])
  zf = jnp.zeros((pad,), _f32)
  cols = [src.astype(_f32), dst.astype(_f32), bond.astype(_f32)]
  cols += [jnp.concatenate([jnp.asarray(edge_attr[:, j], _f32), zf])
           for j in range(K)]
  cols += [jnp.concatenate([jnp.asarray(peripheral_attr[:, j], _f32), zf])
           for j in range(K)]
  epack = (jnp.stack(cols, 0).reshape(11, NW * NCHW, CP)
           .transpose(1, 0, 2).reshape(-1))
  epack = jnp.concatenate([epack, jnp.zeros((FPC,), _f32)])
  z2 = z.astype(jnp.int32).reshape(N, 1)
  batch2 = batch.astype(jnp.int32).reshape(N, 1)
  zero_nh = jnp.zeros((N, H), _f32)
  zero_n4 = jnp.zeros((N * 4,), _f32)

  h0 = _embed(z2, x, rd, z_table, rd_W, rd_b.reshape(1, 8),
              init_W[:8, :], init_W[8:, :], init_b.reshape(1, H))

  vn = jnp.zeros((G, H), _f32)
  h_list = [h0]
  out = None
  for l in range(L):
    k = min(l + 1, K)
    hops = [h_list[l - j] for j in range(k)]
    agg2, w2flat = _EDGE_KERNELS[k](epack, *hops, zero_nh, zero_n4)
    w2 = w2flat.reshape(NC, N, 4)
    scale_row = jnp.full((1, H), 1.0, _f32) + gnn_eps[l]
    args = (h_list[l], agg2, w2, bond_table[l], scale_row,
            gnn_W1[l], gnn_b1[l].reshape(1, 2 * H),
            gnn_W2[l], gnn_b2[l].reshape(1, H))
    if l < L - 1:
      vn_new = _vn_update(h_list[l], vn, batch2,
                          vn_W1[l], vn_b1[l].reshape(1, 2 * H),
                          vn_W2[l], vn_b2[l].reshape(1, H))
      h_list.append(_mlp_mid(*args, vn_new, batch2))
      vn = vn_new
    else:
      out = _mlp_last(*args, out_W, out_b.reshape(1, H))
  return out


# speedup vs baseline: 1.3107x; 1.3107x over previous
import functools

import jax
import jax.numpy as jnp
from jax import lax
from jax.experimental import pallas as pl
from jax.experimental.pallas import tpu as pltpu
from jax.experimental.pallas import tpu_sc as plsc

N = 10000
E = 320000
H = 128
L = 4
K = 4
G = 512
NZ = 1000

NC = 2
NS = 16
NW = NC * NS
CP = 32
EPWP = 10048
EP = NW * EPWP
NCHW = EPWP // CP
FPC = 11 * CP
RPT = 624

R = 2000
GRID_N = N // R

_f32 = jnp.float32
_HOPS = H // 16


def _make_edge_kernel(k):
  mesh = plsc.VectorSubcoreMesh(core_axis_name="c", subcore_axis_name="s")
  out_type = [
      jax.ShapeDtypeStruct((NC, N, H), _f32),
      jax.ShapeDtypeStruct((NC, N * 4), _f32),
  ]
  scratch_types = (
      [pltpu.VMEM((2 * FPC,), _f32)]
      + [pltpu.VMEM((CP,), jnp.int32) for _ in range(4)]
      + [pltpu.VMEM((CP,), _f32)]
      + [pltpu.VMEM((CP, H), _f32) for _ in range(2 * k)]
      + [pltpu.VMEM((CP, H), _f32)]
      + [pltpu.VMEM_SHARED((N, H), _f32),
         pltpu.VMEM_SHARED((N * 4,), _f32)]
      + [pltpu.SemaphoreType.DMA for _ in range(3)]
  )

  @functools.partial(pl.kernel, mesh=mesh, out_type=out_type,
                     scratch_types=scratch_types)
  def edge_kernel(*refs):
    epack = refs[0]
    h_hs = refs[1:1 + k]
    zero_nh = refs[1 + k]
    zero_n4 = refs[2 + k]
    agg_out = refs[3 + k]
    w_out = refs[4 + k]
    s = 5 + k
    ebuf = refs[s]
    src_i = refs[s + 1:s + 3]
    dst_i = refs[s + 3]
    cidx_i = refs[s + 4]
    pbuf = refs[s + 5]
    gb = refs[s + 6:s + 6 + 2 * k]
    g = [gb[:k], gb[k:]]
    msg = refs[s + 6 + 2 * k]
    agg_sh = refs[s + 7 + 2 * k]
    w_sh = refs[s + 8 + 2 * k]
    sems = refs[s + 9 + 2 * k:]
    lsem, gsems = sems[0], sems[1:3]

    cid = lax.axis_index("c")
    sid = lax.axis_index("s")
    base = (cid * NS + sid) * NCHW

    @pl.when(sid == 0)
    def _init():
      pltpu.sync_copy(zero_nh, agg_sh)
      pltpu.sync_copy(zero_n4, w_sh)

    plsc.subcore_barrier()

    def lin_start(c2):
      pltpu.async_copy(epack.at[pl.ds((base + 2 * c2) * FPC, 2 * FPC)], ebuf,
                       lsem)

    def lin_wait(c2):
      pltpu.make_async_copy(epack.at[pl.ds((base + 2 * c2) * FPC, 2 * FPC)],
                            ebuf, lsem).wait()

    def decode_src(p):
      for t in range(CP // 16):
        src_i[p][pl.ds(16 * t, 16)] = ebuf[
            pl.ds(p * FPC + 16 * t, 16)].astype(jnp.int32)

    def gat_start(p):
      for j in range(k):
        pltpu.async_copy(h_hs[j].at[src_i[p]], g[p][j], gsems[p])

    def gat_wait(p):
      for j in range(k):
        pltpu.make_async_copy(h_hs[j].at[src_i[p]], g[p][j], gsems[p]).wait()

    def compute_and_scatter(p):
      o = p * FPC
      for t in range(CP // 16):
        sl = pl.ds(16 * t, 16)
        d = ebuf[pl.ds(o + CP + 16 * t, 16)].astype(jnp.int32)
        b = ebuf[pl.ds(o + 2 * CP + 16 * t, 16)].astype(jnp.int32)
        dst_i[sl] = d
        cidx_i[sl] = d * 4 + b
        pv = ebuf[pl.ds(o + 7 * CP + 16 * t, 16)]
        for j in range(1, k):
          pv = pv + ebuf[pl.ds(o + (7 + j) * CP + 16 * t, 16)]
        pbuf[sl] = pv

      def grp(gi, carry):
        e0 = gi * 16
        ea_vecs = [ebuf[pl.ds(o + (3 + j) * CP + e0, 16)] for j in range(k)]
        for lane in range(16):
          e = e0 + lane
          s0 = ea_vecs[0][lane]
          acc = [g[p][0][e, pl.ds(16 * i, 16)] * s0 for i in range(_HOPS)]
          for j in range(1, k):
            sj = ea_vecs[j][lane]
            for i in range(_HOPS):
              acc[i] = acc[i] + g[p][j][e, pl.ds(16 * i, 16)] * sj
          for i in range(_HOPS):
            msg[e, pl.ds(16 * i, 16)] = acc[i]
        return carry

      lax.fori_loop(0, CP // 16, grp, 0)
      pltpu.sync_copy(msg, agg_sh.at[dst_i], add=True)
      pltpu.sync_copy(pbuf, w_sh.at[cidx_i], add=True)

    def pair(c2, carry):
      lin_start(c2)
      lin_wait(c2)
      decode_src(0)
      gat_start(0)
      decode_src(1)
      gat_start(1)
      gat_wait(0)
      compute_and_scatter(0)
      gat_wait(1)
      compute_and_scatter(1)
      return carry

    lax.fori_loop(0, NCHW // 2, pair, 0)

    plsc.subcore_barrier()
    r0 = sid * RPT
    pltpu.sync_copy(agg_sh.at[pl.ds(r0, RPT)],
                    agg_out.at[cid, pl.ds(r0, RPT)])

    @pl.when(sid == 0)
    def _wout():
      pltpu.sync_copy(agg_sh.at[pl.ds(NS * RPT, N - NS * RPT)],
                      agg_out.at[cid, pl.ds(NS * RPT, N - NS * RPT)])
      pltpu.sync_copy(w_sh, w_out.at[cid])

  return edge_kernel


_EDGE_KERNELS = {k: _make_edge_kernel(k) for k in range(1, K + 1)}


def _embed_body(z_ref, x_ref, rd_ref, zt_ref, rdw_ref, rdb_ref, iwt_ref,
                iwb_ref, ib_ref, o_ref):
  oh = (z_ref[...] == lax.broadcasted_iota(jnp.int32, (1, NZ), 1)).astype(_f32)
  ze = jnp.dot(oh, zt_ref[...], preferred_element_type=_f32)
  a8 = ze + rd_ref[...] * rdw_ref[...] + rdb_ref[...]
  o_ref[...] = (jnp.dot(a8, iwt_ref[...], preferred_element_type=_f32)
                + jnp.dot(x_ref[...], iwb_ref[...], preferred_element_type=_f32)
                + ib_ref[...])


def _embed(z2, x, rd, z_table, rd_W, rd_b2, init_Wt, init_Wb, init_b2):
  return pl.pallas_call(
      _embed_body,
      grid=(GRID_N,),
      in_specs=[
          pl.BlockSpec((R, 1), lambda i: (i, 0)),
          pl.BlockSpec((R, H - 8), lambda i: (i, 0)),
          pl.BlockSpec((R, 1), lambda i: (i, 0)),
          pl.BlockSpec((NZ, 8), lambda i: (0, 0)),
          pl.BlockSpec((1, 8), lambda i: (0, 0)),
          pl.BlockSpec((1, 8), lambda i: (0, 0)),
          pl.BlockSpec((8, H), lambda i: (0, 0)),
          pl.BlockSpec((H - 8, H), lambda i: (0, 0)),
          pl.BlockSpec((1, H), lambda i: (0, 0)),
      ],
      out_specs=pl.BlockSpec((R, H), lambda i: (i, 0)),
      out_shape=jax.ShapeDtypeStruct((N, H), _f32),
  )(z2, x, rd, z_table, rd_W, rd_b2, init_Wt, init_Wb, init_b2)


def _mlp_common(h_ref, agg_ref, w_ref, btab_ref, scale_ref, w1_ref, b1_ref,
                w2_ref, b2_ref):
  a = h_ref[...] * scale_ref[...] + agg_ref[0] + agg_ref[1]
  wsum = w_ref[0] + w_ref[1]
  a = a + jnp.dot(wsum, btab_ref[...], preferred_element_type=_f32)
  t = jnp.maximum(jnp.dot(a, w1_ref[...], preferred_element_type=_f32)
                  + b1_ref[...], 0.0)
  zz = jnp.dot(t, w2_ref[...], preferred_element_type=_f32) + b2_ref[...]
  mu = jnp.mean(zz, axis=-1, keepdims=True)
  var = jnp.mean((zz - mu) ** 2, axis=-1, keepdims=True)
  return (zz - mu) * lax.rsqrt(var + 1e-5)


def _mlp_mid_body(h_ref, agg_ref, w_ref, btab_ref, scale_ref, w1_ref, b1_ref,
                  w2_ref, b2_ref, vn_ref, batch_ref, o_ref):
  ln = _mlp_common(h_ref, agg_ref, w_ref, btab_ref, scale_ref, w1_ref, b1_ref,
                   w2_ref, b2_ref)
  oh = (batch_ref[...] == lax.broadcasted_iota(jnp.int32, (1, G), 1)
        ).astype(_f32)
  o_ref[...] = ln + jnp.dot(oh, vn_ref[...], preferred_element_type=_f32)


def _mlp_last_body(h_ref, agg_ref, w_ref, btab_ref, scale_ref, w1_ref, b1_ref,
                   w2_ref, b2_ref, ow_ref, ob_ref, o_ref):
  ln = _mlp_common(h_ref, agg_ref, w_ref, btab_ref, scale_ref, w1_ref, b1_ref,
                   w2_ref, b2_ref)
  o_ref[...] = jnp.maximum(
      jnp.dot(ln, ow_ref[...], preferred_element_type=_f32) + ob_ref[...], 0.0)


def _mlp_specs(extra):
  return [
      pl.BlockSpec((R, H), lambda i: (i, 0)),
      pl.BlockSpec((NC, R, H), lambda i: (0, i, 0)),
      pl.BlockSpec((NC, R, 4), lambda i: (0, i, 0)),
      pl.BlockSpec((4, H), lambda i: (0, 0)),
      pl.BlockSpec((1, H), lambda i: (0, 0)),
      pl.BlockSpec((H, 2 * H), lambda i: (0, 0)),
      pl.BlockSpec((1, 2 * H), lambda i: (0, 0)),
      pl.BlockSpec((2 * H, H), lambda i: (0, 0)),
      pl.BlockSpec((1, H), lambda i: (0, 0)),
  ] + extra


def _mlp_mid(h, agg2, w2, btab, scale_row, w1, b1, w2w, b2, vn, batch2):
  return pl.pallas_call(
      _mlp_mid_body,
      grid=(GRID_N,),
      in_specs=_mlp_specs([
          pl.BlockSpec((G, H), lambda i: (0, 0)),
          pl.BlockSpec((R, 1), lambda i: (i, 0)),
      ]),
      out_specs=pl.BlockSpec((R, H), lambda i: (i, 0)),
      out_shape=jax.ShapeDtypeStruct((N, H), _f32),
  )(h, agg2, w2, btab, scale_row, w1, b1, w2w, b2, vn, batch2)


def _mlp_last(h, agg2, w2, btab, scale_row, w1, b1, w2w, b2, out_W, out_b2):
  return pl.pallas_call(
      _mlp_last_body,
      grid=(GRID_N,),
      in_specs=_mlp_specs([
          pl.BlockSpec((H, H), lambda i: (0, 0)),
          pl.BlockSpec((1, H), lambda i: (0, 0)),
      ]),
      out_specs=pl.BlockSpec((R, H), lambda i: (i, 0)),
      out_shape=jax.ShapeDtypeStruct((N, H), _f32),
  )(h, agg2, w2, btab, scale_row, w1, b1, w2w, b2, out_W, out_b2)


def _vn_body(h_ref, vno_ref, batch_ref, w1_ref, b1_ref, w2_ref, b2_ref, o_ref):
  step = pl.program_id(0)
  oh = (batch_ref[...] == lax.broadcasted_iota(jnp.int32, (1, G), 1)
        ).astype(_f32)
  seg = lax.dot_general(oh, h_ref[...], (((0,), (0,)), ((), ())),
                        preferred_element_type=_f32)

  @pl.when(step == 0)
  def _():
    o_ref[...] = vno_ref[...] + seg

  @pl.when(step > 0)
  def _():
    o_ref[...] = o_ref[...] + seg

  @pl.when(step == GRID_N - 1)
  def _():
    acc = o_ref[...]
    t = jnp.maximum(jnp.dot(acc, w1_ref[...], preferred_element_type=_f32)
                    + b1_ref[...], 0.0)
    o_ref[...] = jnp.maximum(
        jnp.dot(t, w2_ref[...], preferred_element_type=_f32) + b2_ref[...], 0.0)


def _vn_update(h, vn_old, batch2, w1, b1, w2, b2):
  return pl.pallas_call(
      _vn_body,
      grid=(GRID_N,),
      in_specs=[
          pl.BlockSpec((R, H), lambda i: (i, 0)),
          pl.BlockSpec((G, H), lambda i: (0, 0)),
          pl.BlockSpec((R, 1), lambda i: (i, 0)),
          pl.BlockSpec((H, 2 * H), lambda i: (0, 0)),
          pl.BlockSpec((1, 2 * H), lambda i: (0, 0)),
          pl.BlockSpec((2 * H, H), lambda i: (0, 0)),
          pl.BlockSpec((1, H), lambda i: (0, 0)),
      ],
      out_specs=pl.BlockSpec((G, H), lambda i: (0, 0)),
      out_shape=jax.ShapeDtypeStruct((G, H), _f32),
  )(h, vn_old, batch2, w1, b1, w2, b2)


def kernel(x, z, edge_index, bond_feature, edge_attr, peripheral_attr, rd,
           pos, batch, z_table, rd_W, rd_b, init_W, init_b, gnn_W1, gnn_b1,
           gnn_W2, gnn_b2, gnn_eps, bond_table, vn_W1, vn_b1, vn_W2, vn_b2,
           out_W, out_b):
  del pos
  pad = EP - E
  pad_ids = jnp.arange(pad, dtype=jnp.int32) % N
  src = jnp.concatenate([edge_index[0].astype(jnp.int32), pad_ids])
  dst = jnp.concatenate([edge_index[1].astype(jnp.int32), pad_ids])
  bond = jnp.concatenate([bond_feature.astype(jnp.int32),
                          jnp.zeros((pad,), jnp.int32)])
  zf = jnp.zeros((pad,), _f32)
  cols = [src.astype(_f32), dst.astype(_f32), bond.astype(_f32)]
  cols += [jnp.concatenate([jnp.asarray(edge_attr[:, j], _f32), zf])
           for j in range(K)]
  cols += [jnp.concatenate([jnp.asarray(peripheral_attr[:, j], _f32), zf])
           for j in range(K)]
  epack = (jnp.stack(cols, 0).reshape(11, NW * NCHW, CP)
           .transpose(1, 0, 2).reshape(-1))
  epack = jnp.concatenate([epack, jnp.zeros((FPC,), _f32)])
  z2 = z.astype(jnp.int32).reshape(N, 1)
  batch2 = batch.astype(jnp.int32).reshape(N, 1)
  zero_nh = jnp.zeros((N, H), _f32)
  zero_n4 = jnp.zeros((N * 4,), _f32)

  h0 = _embed(z2, x, rd, z_table, rd_W, rd_b.reshape(1, 8),
              init_W[:8, :], init_W[8:, :], init_b.reshape(1, H))

  vn = jnp.zeros((G, H), _f32)
  h_list = [h0]
  out = None
  for l in range(L):
    k = min(l + 1, K)
    hops = [h_list[l - j] for j in range(k)]
    agg2, w2flat = _EDGE_KERNELS[k](epack, *hops, zero_nh, zero_n4)
    w2 = w2flat.reshape(NC, N, 4)
    scale_row = jnp.full((1, H), 1.0, _f32) + gnn_eps[l]
    args = (h_list[l], agg2, w2, bond_table[l], scale_row,
            gnn_W1[l], gnn_b1[l].reshape(1, 2 * H),
            gnn_W2[l], gnn_b2[l].reshape(1, H))
    if l < L - 1:
      vn_new = _vn_update(h_list[l], vn, batch2,
                          vn_W1[l], vn_b1[l].reshape(1, 2 * H),
                          vn_W2[l], vn_b2[l].reshape(1, H))
      h_list.append(_mlp_mid(*args, vn_new, batch2))
      vn = vn_new
    else:
      out = _mlp_last(*args, out_W, out_b.reshape(1, H))
  return out


# speedup vs baseline: 1.3287x; 1.0138x over previous
import functools

import jax
import jax.numpy as jnp
from jax import lax
from jax.experimental import pallas as pl
from jax.experimental.pallas import tpu as pltpu
from jax.experimental.pallas import tpu_sc as plsc

N = 10000
E = 320000
H = 128
L = 4
K = 4
G = 512
NZ = 1000

NC = 2
NS = 16
NW = NC * NS
CP = 32
EPWP = 10048
EP = NW * EPWP
NCHW = EPWP // CP
FPC = 11 * CP
RPT = 624

R = 2000
GRID_N = N // R

_f32 = jnp.float32
_HOPS = H // 16


def _make_edge_kernel(k):
  mesh = plsc.VectorSubcoreMesh(core_axis_name="c", subcore_axis_name="s")
  out_type = [
      jax.ShapeDtypeStruct((NC, N, H), _f32),
      jax.ShapeDtypeStruct((NC, N * 4), _f32),
  ]
  scratch_types = (
      [pltpu.VMEM((2 * FPC,), _f32)]
      + [pltpu.VMEM((CP,), jnp.int32) for _ in range(3)]
      + [pltpu.VMEM((2 * CP,), jnp.int32)]
      + [pltpu.VMEM((2 * CP,), _f32)]
      + [pltpu.VMEM((CP, H), _f32) for _ in range(2 * k)]
      + [pltpu.VMEM((CP, H), _f32)]
      + [pltpu.VMEM_SHARED((N, H), _f32),
         pltpu.VMEM_SHARED((N * 4,), _f32)]
      + [pltpu.SemaphoreType.DMA for _ in range(3)]
  )

  @functools.partial(pl.kernel, mesh=mesh, out_type=out_type,
                     scratch_types=scratch_types)
  def edge_kernel(*refs):
    epack = refs[0]
    h_hs = refs[1:1 + k]
    zero_nh = refs[1 + k]
    zero_n4 = refs[2 + k]
    agg_out = refs[3 + k]
    w_out = refs[4 + k]
    s = 5 + k
    ebuf = refs[s]
    src_i = refs[s + 1:s + 3]
    dst_i = refs[s + 3]
    cidx_i = refs[s + 4]
    pbuf = refs[s + 5]

    gb = refs[s + 6:s + 6 + 2 * k]
    g = [gb[:k], gb[k:]]
    msg = refs[s + 6 + 2 * k]
    agg_sh = refs[s + 7 + 2 * k]
    w_sh = refs[s + 8 + 2 * k]
    sems = refs[s + 9 + 2 * k:]
    lsem, gsems = sems[0], sems[1:3]

    cid = lax.axis_index("c")
    sid = lax.axis_index("s")
    base = (cid * NS + sid) * NCHW

    @pl.when(sid == 0)
    def _init():
      pltpu.sync_copy(zero_nh, agg_sh)
      pltpu.sync_copy(zero_n4, w_sh)

    plsc.subcore_barrier()

    def lin_start(c2):
      pltpu.async_copy(epack.at[pl.ds((base + 2 * c2) * FPC, 2 * FPC)], ebuf,
                       lsem)

    def lin_wait(c2):
      pltpu.make_async_copy(epack.at[pl.ds((base + 2 * c2) * FPC, 2 * FPC)],
                            ebuf, lsem).wait()

    def decode_src(p):
      for t in range(CP // 16):
        src_i[p][pl.ds(16 * t, 16)] = ebuf[
            pl.ds(p * FPC + 16 * t, 16)].astype(jnp.int32)

    def gat_start(p):
      for j in range(k):
        pltpu.async_copy(h_hs[j].at[src_i[p]], g[p][j], gsems[p])

    def gat_wait(p):
      for j in range(k):
        pltpu.make_async_copy(h_hs[j].at[src_i[p]], g[p][j], gsems[p]).wait()

    def compute_and_scatter(p):
      o = p * FPC
      for t in range(CP // 16):
        sl = pl.ds(16 * t, 16)
        sl2 = pl.ds(p * CP + 16 * t, 16)
        d = ebuf[pl.ds(o + CP + 16 * t, 16)].astype(jnp.int32)
        b = ebuf[pl.ds(o + 2 * CP + 16 * t, 16)].astype(jnp.int32)
        dst_i[sl] = d
        cidx_i[sl2] = d * 4 + b
        pv = ebuf[pl.ds(o + 7 * CP + 16 * t, 16)]
        for j in range(1, k):
          pv = pv + ebuf[pl.ds(o + (7 + j) * CP + 16 * t, 16)]
        pbuf[sl2] = pv

      def grp(gi, carry):
        e0 = gi * 16
        ea_vecs = [ebuf[pl.ds(o + (3 + j) * CP + e0, 16)] for j in range(k)]
        for lane in range(16):
          e = e0 + lane
          s0 = ea_vecs[0][lane]
          acc = [g[p][0][e, pl.ds(16 * i, 16)] * s0 for i in range(_HOPS)]
          for j in range(1, k):
            sj = ea_vecs[j][lane]
            for i in range(_HOPS):
              acc[i] = acc[i] + g[p][j][e, pl.ds(16 * i, 16)] * sj
          for i in range(_HOPS):
            msg[e, pl.ds(16 * i, 16)] = acc[i]
        return carry

      lax.fori_loop(0, CP // 16, grp, 0)
      pltpu.sync_copy(msg, agg_sh.at[dst_i], add=True)

    def pair(c2, carry):
      lin_start(c2)
      lin_wait(c2)
      decode_src(0)
      gat_start(0)
      decode_src(1)
      gat_start(1)
      gat_wait(0)
      compute_and_scatter(0)
      gat_wait(1)
      compute_and_scatter(1)
      pltpu.sync_copy(pbuf, w_sh.at[cidx_i], add=True)
      return carry

    lax.fori_loop(0, NCHW // 2, pair, 0)

    plsc.subcore_barrier()
    r0 = sid * RPT
    pltpu.sync_copy(agg_sh.at[pl.ds(r0, RPT)],
                    agg_out.at[cid, pl.ds(r0, RPT)])

    @pl.when(sid == 0)
    def _wout():
      pltpu.sync_copy(agg_sh.at[pl.ds(NS * RPT, N - NS * RPT)],
                      agg_out.at[cid, pl.ds(NS * RPT, N - NS * RPT)])
      pltpu.sync_copy(w_sh, w_out.at[cid])

  return edge_kernel


_EDGE_KERNELS = {k: _make_edge_kernel(k) for k in range(1, K + 1)}


def _embed_body(z_ref, x_ref, rd_ref, zt_ref, rdw_ref, rdb_ref, iwt_ref,
                iwb_ref, ib_ref, o_ref):
  oh = (z_ref[...] == lax.broadcasted_iota(jnp.int32, (1, NZ), 1)).astype(_f32)
  ze = jnp.dot(oh, zt_ref[...], preferred_element_type=_f32)
  a8 = ze + rd_ref[...] * rdw_ref[...] + rdb_ref[...]
  o_ref[...] = (jnp.dot(a8, iwt_ref[...], preferred_element_type=_f32)
                + jnp.dot(x_ref[...], iwb_ref[...], preferred_element_type=_f32)
                + ib_ref[...])


def _embed(z2, x, rd, z_table, rd_W, rd_b2, init_Wt, init_Wb, init_b2):
  return pl.pallas_call(
      _embed_body,
      grid=(GRID_N,),
      in_specs=[
          pl.BlockSpec((R, 1), lambda i: (i, 0)),
          pl.BlockSpec((R, H - 8), lambda i: (i, 0)),
          pl.BlockSpec((R, 1), lambda i: (i, 0)),
          pl.BlockSpec((NZ, 8), lambda i: (0, 0)),
          pl.BlockSpec((1, 8), lambda i: (0, 0)),
          pl.BlockSpec((1, 8), lambda i: (0, 0)),
          pl.BlockSpec((8, H), lambda i: (0, 0)),
          pl.BlockSpec((H - 8, H), lambda i: (0, 0)),
          pl.BlockSpec((1, H), lambda i: (0, 0)),
      ],
      out_specs=pl.BlockSpec((R, H), lambda i: (i, 0)),
      out_shape=jax.ShapeDtypeStruct((N, H), _f32),
  )(z2, x, rd, z_table, rd_W, rd_b2, init_Wt, init_Wb, init_b2)


def _mlp_common(h_ref, agg_ref, w_ref, btab_ref, scale_ref, w1_ref, b1_ref,
                w2_ref, b2_ref):
  a = h_ref[...] * scale_ref[...] + agg_ref[0] + agg_ref[1]
  wsum = w_ref[0] + w_ref[1]
  a = a + jnp.dot(wsum, btab_ref[...], preferred_element_type=_f32)
  t = jnp.maximum(jnp.dot(a, w1_ref[...], preferred_element_type=_f32)
                  + b1_ref[...], 0.0)
  zz = jnp.dot(t, w2_ref[...], preferred_element_type=_f32) + b2_ref[...]
  mu = jnp.mean(zz, axis=-1, keepdims=True)
  var = jnp.mean((zz - mu) ** 2, axis=-1, keepdims=True)
  return (zz - mu) * lax.rsqrt(var + 1e-5)


def _mlp_mid_body(h_ref, agg_ref, w_ref, btab_ref, scale_ref, w1_ref, b1_ref,
                  w2_ref, b2_ref, vn_ref, batch_ref, o_ref):
  ln = _mlp_common(h_ref, agg_ref, w_ref, btab_ref, scale_ref, w1_ref, b1_ref,
                   w2_ref, b2_ref)
  oh = (batch_ref[...] == lax.broadcasted_iota(jnp.int32, (1, G), 1)
        ).astype(_f32)
  o_ref[...] = ln + jnp.dot(oh, vn_ref[...], preferred_element_type=_f32)


def _mlp_last_body(h_ref, agg_ref, w_ref, btab_ref, scale_ref, w1_ref, b1_ref,
                   w2_ref, b2_ref, ow_ref, ob_ref, o_ref):
  ln = _mlp_common(h_ref, agg_ref, w_ref, btab_ref, scale_ref, w1_ref, b1_ref,
                   w2_ref, b2_ref)
  o_ref[...] = jnp.maximum(
      jnp.dot(ln, ow_ref[...], preferred_element_type=_f32) + ob_ref[...], 0.0)


def _mlp_specs(extra):
  return [
      pl.BlockSpec((R, H), lambda i: (i, 0)),
      pl.BlockSpec((NC, R, H), lambda i: (0, i, 0)),
      pl.BlockSpec((NC, R, 4), lambda i: (0, i, 0)),
      pl.BlockSpec((4, H), lambda i: (0, 0)),
      pl.BlockSpec((1, H), lambda i: (0, 0)),
      pl.BlockSpec((H, 2 * H), lambda i: (0, 0)),
      pl.BlockSpec((1, 2 * H), lambda i: (0, 0)),
      pl.BlockSpec((2 * H, H), lambda i: (0, 0)),
      pl.BlockSpec((1, H), lambda i: (0, 0)),
  ] + extra


def _mlp_mid(h, agg2, w2, btab, scale_row, w1, b1, w2w, b2, vn, batch2):
  return pl.pallas_call(
      _mlp_mid_body,
      grid=(GRID_N,),
      in_specs=_mlp_specs([
          pl.BlockSpec((G, H), lambda i: (0, 0)),
          pl.BlockSpec((R, 1), lambda i: (i, 0)),
      ]),
      out_specs=pl.BlockSpec((R, H), lambda i: (i, 0)),
      out_shape=jax.ShapeDtypeStruct((N, H), _f32),
  )(h, agg2, w2, btab, scale_row, w1, b1, w2w, b2, vn, batch2)


def _mlp_last(h, agg2, w2, btab, scale_row, w1, b1, w2w, b2, out_W, out_b2):
  return pl.pallas_call(
      _mlp_last_body,
      grid=(GRID_N,),
      in_specs=_mlp_specs([
          pl.BlockSpec((H, H), lambda i: (0, 0)),
          pl.BlockSpec((1, H), lambda i: (0, 0)),
      ]),
      out_specs=pl.BlockSpec((R, H), lambda i: (i, 0)),
      out_shape=jax.ShapeDtypeStruct((N, H), _f32),
  )(h, agg2, w2, btab, scale_row, w1, b1, w2w, b2, out_W, out_b2)


def _vn_body(h_ref, vno_ref, batch_ref, w1_ref, b1_ref, w2_ref, b2_ref, o_ref):
  step = pl.program_id(0)
  oh = (batch_ref[...] == lax.broadcasted_iota(jnp.int32, (1, G), 1)
        ).astype(_f32)
  seg = lax.dot_general(oh, h_ref[...], (((0,), (0,)), ((), ())),
                        preferred_element_type=_f32)

  @pl.when(step == 0)
  def _():
    o_ref[...] = vno_ref[...] + seg

  @pl.when(step > 0)
  def _():
    o_ref[...] = o_ref[...] + seg

  @pl.when(step == GRID_N - 1)
  def _():
    acc = o_ref[...]
    t = jnp.maximum(jnp.dot(acc, w1_ref[...], preferred_element_type=_f32)
                    + b1_ref[...], 0.0)
    o_ref[...] = jnp.maximum(
        jnp.dot(t, w2_ref[...], preferred_element_type=_f32) + b2_ref[...], 0.0)


def _vn_update(h, vn_old, batch2, w1, b1, w2, b2):
  return pl.pallas_call(
      _vn_body,
      grid=(GRID_N,),
      in_specs=[
          pl.BlockSpec((R, H), lambda i: (i, 0)),
          pl.BlockSpec((G, H), lambda i: (0, 0)),
          pl.BlockSpec((R, 1), lambda i: (i, 0)),
          pl.BlockSpec((H, 2 * H), lambda i: (0, 0)),
          pl.BlockSpec((1, 2 * H), lambda i: (0, 0)),
          pl.BlockSpec((2 * H, H), lambda i: (0, 0)),
          pl.BlockSpec((1, H), lambda i: (0, 0)),
      ],
      out_specs=pl.BlockSpec((G, H), lambda i: (0, 0)),
      out_shape=jax.ShapeDtypeStruct((G, H), _f32),
  )(h, vn_old, batch2, w1, b1, w2, b2)


def kernel(x, z, edge_index, bond_feature, edge_attr, peripheral_attr, rd,
           pos, batch, z_table, rd_W, rd_b, init_W, init_b, gnn_W1, gnn_b1,
           gnn_W2, gnn_b2, gnn_eps, bond_table, vn_W1, vn_b1, vn_W2, vn_b2,
           out_W, out_b):
  del pos
  pad = EP - E
  pad_ids = jnp.arange(pad, dtype=jnp.int32) % N
  src = jnp.concatenate([edge_index[0].astype(jnp.int32), pad_ids])
  dst = jnp.concatenate([edge_index[1].astype(jnp.int32), pad_ids])
  bond = jnp.concatenate([bond_feature.astype(jnp.int32),
                          jnp.zeros((pad,), jnp.int32)])
  zf = jnp.zeros((pad,), _f32)
  cols = [src.astype(_f32), dst.astype(_f32), bond.astype(_f32)]
  cols += [jnp.concatenate([jnp.asarray(edge_attr[:, j], _f32), zf])
           for j in range(K)]
  cols += [jnp.concatenate([jnp.asarray(peripheral_attr[:, j], _f32), zf])
           for j in range(K)]
  epack = (jnp.stack(cols, 0).reshape(11, NW * NCHW, CP)
           .transpose(1, 0, 2).reshape(-1))
  epack = jnp.concatenate([epack, jnp.zeros((FPC,), _f32)])
  z2 = z.astype(jnp.int32).reshape(N, 1)
  batch2 = batch.astype(jnp.int32).reshape(N, 1)
  zero_nh = jnp.zeros((N, H), _f32)
  zero_n4 = jnp.zeros((N * 4,), _f32)

  h0 = _embed(z2, x, rd, z_table, rd_W, rd_b.reshape(1, 8),
              init_W[:8, :], init_W[8:, :], init_b.reshape(1, H))

  vn = jnp.zeros((G, H), _f32)
  h_list = [h0]
  out = None
  for l in range(L):
    k = min(l + 1, K)
    hops = [h_list[l - j] for j in range(k)]
    agg2, w2flat = _EDGE_KERNELS[k](epack, *hops, zero_nh, zero_n4)
    w2 = w2flat.reshape(NC, N, 4)
    scale_row = jnp.full((1, H), 1.0, _f32) + gnn_eps[l]
    args = (h_list[l], agg2, w2, bond_table[l], scale_row,
            gnn_W1[l], gnn_b1[l].reshape(1, 2 * H),
            gnn_W2[l], gnn_b2[l].reshape(1, H))
    if l < L - 1:
      vn_new = _vn_update(h_list[l], vn, batch2,
                          vn_W1[l], vn_b1[l].reshape(1, 2 * H),
                          vn_W2[l], vn_b2[l].reshape(1, H))
      h_list.append(_mlp_mid(*args, vn_new, batch2))
      vn = vn_new
    else:
      out = _mlp_last(*args, out_W, out_b.reshape(1, H))
  return out
